# baseline (speedup 1.0000x reference)
import jax
import jax.numpy as jnp
from jax.experimental import pallas as pl
from jax.experimental.pallas import tpu as pltpu

_LEVELS = 4
_CODES = 1024
_DIM = 64
_COMMIT = 0.25
_BLOCK = 256
_BIG = 2 ** 30


def _foldsum(s):
    acc = s[0:8, :]
    for j in range(1, 8):
        acc = acc + s[8 * j:8 * j + 8, :]
    acc = acc[0:4, :] + acc[4:8, :]
    acc = acc[0:2, :] + acc[2:4, :]
    return acc[0:1, :] + acc[1:2, :]


def _rvq_kernel(x_ref, emb_ref, embT_ref, q_ref, idx_ref, loss_ref,
                esqb_ref, iota_ref, hi_ref, mid_ref, lo_ref):
    i = pl.program_id(0)

    @pl.when(i == 0)
    def _init():
        for lvl in range(_LEVELS):
            eT = embT_ref[lvl]
            esq_row = _foldsum(eT * eT)
            esqb_ref[lvl] = jnp.broadcast_to(
                esq_row.reshape(_CODES, 1), (_CODES, _BLOCK))
            hi = eT.astype(jnp.bfloat16)
            rem = eT - hi.astype(jnp.float32)
            mid = rem.astype(jnp.bfloat16)
            lo = (rem - mid.astype(jnp.float32)).astype(jnp.bfloat16)
            hi_ref[lvl] = hi
            mid_ref[lvl] = mid
            lo_ref[lvl] = lo
        iota_ref[...] = jax.lax.broadcasted_iota(
            jnp.int32, (_CODES, _BLOCK), 0)
        loss_ref[...] = jnp.zeros_like(loss_ref)

    xT = x_ref[...].T
    rowids = iota_ref[...]
    res = xT
    qs = jnp.zeros_like(xT)
    rsq = _foldsum(res * res)
    level_idx = []
    level_loss = []
    for lvl in range(_LEVELS):
        emb = emb_ref[lvl]
        prodT = jax.lax.dot_general(
            emb, res, (((1,), (0,)), ((), ())),
            preferred_element_type=jnp.float32)
        d = (esqb_ref[lvl] + rsq) - 2.0 * prodT
        dmin = jnp.min(d, axis=0, keepdims=True)
        idx = jnp.min(jnp.where(d == dmin, rowids, jnp.int32(_BIG)),
                      axis=0, keepdims=True)
        onehot = jnp.where(rowids == idx, jnp.float32(1.0),
                           jnp.float32(0.0)).astype(jnp.bfloat16)
        dn = (((1,), (0,)), ((), ()))
        qlo = jax.lax.dot_general(lo_ref[lvl], onehot, dn,
                                  preferred_element_type=jnp.float32)
        qmid = jax.lax.dot_general(mid_ref[lvl], onehot, dn,
                                   preferred_element_type=jnp.float32)
        qhi = jax.lax.dot_general(hi_ref[lvl], onehot, dn,
                                  preferred_element_type=jnp.float32)
        qT = (qlo + qmid) + qhi
        res = res - qT
        qs = qs + qT
        rsq = _foldsum(res * res)
        level_idx.append(idx)
        level_loss.append(rsq)
    q_ref[...] = (xT + (qs - xT)).T
    idx_ref[...] = jnp.concatenate(level_idx, axis=0)
    loss_ref[...] += jnp.concatenate(level_loss, axis=0)


def kernel(inputs, embedding):
    n, dim = inputs.shape
    grid = n // _BLOCK
    emb_t = jnp.transpose(embedding, (0, 2, 1))
    q, idx, loss = pl.pallas_call(
        _rvq_kernel,
        grid=(grid,),
        in_specs=[
            pl.BlockSpec((_BLOCK, dim), lambda i: (i, 0)),
            pl.BlockSpec((_LEVELS, _CODES, _DIM), lambda i: (0, 0, 0)),
            pl.BlockSpec((_LEVELS, _DIM, _CODES), lambda i: (0, 0, 0)),
        ],
        out_specs=(
            pl.BlockSpec((_BLOCK, dim), lambda i: (i, 0)),
            pl.BlockSpec((_LEVELS, _BLOCK), lambda i: (0, i)),
            pl.BlockSpec((_LEVELS, _BLOCK), lambda i: (0, 0)),
        ),
        out_shape=(
            jax.ShapeDtypeStruct((n, dim), jnp.float32),
            jax.ShapeDtypeStruct((_LEVELS, n), jnp.int32),
            jax.ShapeDtypeStruct((_LEVELS, _BLOCK), jnp.float32),
        ),
        scratch_shapes=[
            pltpu.VMEM((_LEVELS, _CODES, _BLOCK), jnp.float32),
            pltpu.VMEM((_CODES, _BLOCK), jnp.int32),
            pltpu.VMEM((_LEVELS, _DIM, _CODES), jnp.bfloat16),
            pltpu.VMEM((_LEVELS, _DIM, _CODES), jnp.bfloat16),
            pltpu.VMEM((_LEVELS, _DIM, _CODES), jnp.bfloat16),
        ],
    )(inputs, embedding, emb_t)
    denom = jnp.float32(n * dim)
    per_level = jnp.sum(loss, axis=1) / denom
    cb = per_level[0] + per_level[1] + per_level[2] + per_level[3]
    commit = cb
    vq = cb + jnp.float32(_COMMIT) * commit
    return (q, idx, vq, cb, commit)


# speedup vs baseline: 1.1844x; 1.1844x over previous
import jax
import jax.numpy as jnp
from jax.experimental import pallas as pl
from jax.experimental.pallas import tpu as pltpu

_LEVELS = 4
_CODES = 1024
_DIM = 64
_COMMIT = 0.25
_BLOCK = 256
_BIG = 2 ** 30


def _foldsum(s):
    acc = s[0:8, :]
    for j in range(1, 8):
        acc = acc + s[8 * j:8 * j + 8, :]
    acc = acc[0:4, :] + acc[4:8, :]
    acc = acc[0:2, :] + acc[2:4, :]
    return acc[0:1, :] + acc[1:2, :]


def _rvq_kernel(x_ref, emb_ref, embT_ref, q_ref, idx_ref, loss_ref,
                esqb_ref, iota_ref, split_ref):
    i = pl.program_id(0)

    @pl.when(i == 0)
    def _init():
        for lvl in range(_LEVELS):
            eT = embT_ref[lvl]
            esq_row = _foldsum(eT * eT)
            esqb_ref[lvl] = jnp.broadcast_to(
                esq_row.reshape(_CODES, 1), (_CODES, _BLOCK))
            hi = eT.astype(jnp.bfloat16)
            rem = eT - hi.astype(jnp.float32)
            mid = rem.astype(jnp.bfloat16)
            lo = (rem - mid.astype(jnp.float32)).astype(jnp.bfloat16)
            split_ref[lvl] = jnp.concatenate([lo, mid, hi], axis=0)
        iota_ref[...] = jax.lax.broadcasted_iota(
            jnp.int32, (_CODES, _BLOCK), 0)
        loss_ref[...] = jnp.zeros_like(loss_ref)

    xT = x_ref[...].T
    rowids = iota_ref[...]
    res = xT
    qs = jnp.zeros_like(xT)
    rsq = _foldsum(res * res)
    level_idx = []
    level_loss = []
    for lvl in range(_LEVELS):
        emb = emb_ref[lvl]
        prodT = jax.lax.dot_general(
            emb, res, (((1,), (0,)), ((), ())),
            preferred_element_type=jnp.float32)
        d = (esqb_ref[lvl] + rsq) - 2.0 * prodT
        dmin = jnp.min(d, axis=0, keepdims=True)
        idx = jnp.min(jnp.where(d == dmin, rowids, jnp.int32(_BIG)),
                      axis=0, keepdims=True)
        onehot = jnp.where(rowids == idx, jnp.float32(1.0),
                           jnp.float32(0.0)).astype(jnp.bfloat16)
        q3 = jax.lax.dot_general(
            split_ref[lvl], onehot, (((1,), (0,)), ((), ())),
            preferred_element_type=jnp.float32)
        qT = (q3[0:_DIM] + q3[_DIM:2 * _DIM]) + q3[2 * _DIM:3 * _DIM]
        res = res - qT
        qs = qs + qT
        rsq = _foldsum(res * res)
        level_idx.append(idx)
        level_loss.append(rsq)
    q_ref[...] = (xT + (qs - xT)).T
    idx_ref[...] = jnp.concatenate(level_idx, axis=0)
    loss_ref[...] += jnp.concatenate(level_loss, axis=0)


def kernel(inputs, embedding):
    n, dim = inputs.shape
    grid = n // _BLOCK
    emb_t = jnp.transpose(embedding, (0, 2, 1))
    q, idx, loss = pl.pallas_call(
        _rvq_kernel,
        grid=(grid,),
        in_specs=[
            pl.BlockSpec((_BLOCK, dim), lambda i: (i, 0)),
            pl.BlockSpec((_LEVELS, _CODES, _DIM), lambda i: (0, 0, 0)),
            pl.BlockSpec((_LEVELS, _DIM, _CODES), lambda i: (0, 0, 0)),
        ],
        out_specs=(
            pl.BlockSpec((_BLOCK, dim), lambda i: (i, 0)),
            pl.BlockSpec((_LEVELS, _BLOCK), lambda i: (0, i)),
            pl.BlockSpec((_LEVELS, _BLOCK), lambda i: (0, 0)),
        ),
        out_shape=(
            jax.ShapeDtypeStruct((n, dim), jnp.float32),
            jax.ShapeDtypeStruct((_LEVELS, n), jnp.int32),
            jax.ShapeDtypeStruct((_LEVELS, _BLOCK), jnp.float32),
        ),
        scratch_shapes=[
            pltpu.VMEM((_LEVELS, _CODES, _BLOCK), jnp.float32),
            pltpu.VMEM((_CODES, _BLOCK), jnp.int32),
            pltpu.VMEM((_LEVELS, 3 * _DIM, _CODES), jnp.bfloat16),
        ],
    )(inputs, embedding, emb_t)
    denom = jnp.float32(n * dim)
    per_level = jnp.sum(loss, axis=1) / denom
    cb = per_level[0] + per_level[1] + per_level[2] + per_level[3]
    commit = cb
    vq = cb + jnp.float32(_COMMIT) * commit
    return (q, idx, vq, cb, commit)


# speedup vs baseline: 1.1920x; 1.0064x over previous
import jax
import jax.numpy as jnp
from jax.experimental import pallas as pl
from jax.experimental.pallas import tpu as pltpu

_LEVELS = 4
_CODES = 1024
_DIM = 64
_COMMIT = 0.25
_BLOCK = 256
_BIG = 2 ** 30


def _foldsum(s):
    acc = s[0:8, :]
    for j in range(1, 8):
        acc = acc + s[8 * j:8 * j + 8, :]
    acc = acc[0:4, :] + acc[4:8, :]
    acc = acc[0:2, :] + acc[2:4, :]
    return acc[0:1, :] + acc[1:2, :]


def _rvq_kernel(x_ref, emb_ref, embT_ref, q_ref, idx_ref, loss_ref,
                esqb_ref, iota_ref, split_ref, emb2_ref):
    i = pl.program_id(0)

    @pl.when(i == 0)
    def _init():
        for lvl in range(_LEVELS):
            eT = embT_ref[lvl]
            esq_row = _foldsum(eT * eT)
            esqb_ref[lvl] = jnp.broadcast_to(
                esq_row.reshape(_CODES, 1), (_CODES, _BLOCK))
            hi = eT.astype(jnp.bfloat16)
            rem = eT - hi.astype(jnp.float32)
            mid = rem.astype(jnp.bfloat16)
            lo = (rem - mid.astype(jnp.float32)).astype(jnp.bfloat16)
            split_ref[lvl] = jnp.concatenate([lo, mid, hi], axis=0)
            emb2_ref[lvl] = emb_ref[lvl] + emb_ref[lvl]
        iota_ref[...] = jax.lax.broadcasted_iota(
            jnp.int32, (_CODES, _BLOCK), 0)
        loss_ref[...] = jnp.zeros_like(loss_ref)

    xT = x_ref[...].T
    rowids = iota_ref[...]
    res = xT
    qs = jnp.zeros_like(xT)
    rsq = _foldsum(res * res)
    level_idx = []
    level_loss = []
    for lvl in range(_LEVELS):
        prod2 = jax.lax.dot_general(
            emb2_ref[lvl], res, (((1,), (0,)), ((), ())),
            preferred_element_type=jnp.float32)
        d = (esqb_ref[lvl] + rsq) - prod2
        dmin = jnp.min(d, axis=0, keepdims=True)
        idx = jnp.min(jnp.where(d == dmin, rowids, jnp.int32(_BIG)),
                      axis=0, keepdims=True)
        onehot = jnp.where(rowids == idx, jnp.float32(1.0),
                           jnp.float32(0.0)).astype(jnp.bfloat16)
        q3 = jax.lax.dot_general(
            split_ref[lvl], onehot, (((1,), (0,)), ((), ())),
            preferred_element_type=jnp.float32)
        qT = (q3[0:_DIM] + q3[_DIM:2 * _DIM]) + q3[2 * _DIM:3 * _DIM]
        res = res - qT
        qs = qs + qT
        rsq = _foldsum(res * res)
        level_idx.append(idx)
        level_loss.append(rsq)
    q_ref[...] = (xT + (qs - xT)).T
    idx_ref[...] = jnp.concatenate(level_idx, axis=0)
    loss_ref[...] += jnp.concatenate(level_loss, axis=0)


def kernel(inputs, embedding):
    n, dim = inputs.shape
    grid = n // _BLOCK
    emb_t = jnp.transpose(embedding, (0, 2, 1))
    q, idx, loss = pl.pallas_call(
        _rvq_kernel,
        grid=(grid,),
        in_specs=[
            pl.BlockSpec((_BLOCK, dim), lambda i: (i, 0)),
            pl.BlockSpec((_LEVELS, _CODES, _DIM), lambda i: (0, 0, 0)),
            pl.BlockSpec((_LEVELS, _DIM, _CODES), lambda i: (0, 0, 0)),
        ],
        out_specs=(
            pl.BlockSpec((_BLOCK, dim), lambda i: (i, 0)),
            pl.BlockSpec((_LEVELS, _BLOCK), lambda i: (0, i)),
            pl.BlockSpec((_LEVELS, _BLOCK), lambda i: (0, 0)),
        ),
        out_shape=(
            jax.ShapeDtypeStruct((n, dim), jnp.float32),
            jax.ShapeDtypeStruct((_LEVELS, n), jnp.int32),
            jax.ShapeDtypeStruct((_LEVELS, _BLOCK), jnp.float32),
        ),
        scratch_shapes=[
            pltpu.VMEM((_LEVELS, _CODES, _BLOCK), jnp.float32),
            pltpu.VMEM((_CODES, _BLOCK), jnp.int32),
            pltpu.VMEM((_LEVELS, 3 * _DIM, _CODES), jnp.bfloat16),
            pltpu.VMEM((_LEVELS, _CODES, _DIM), jnp.float32),
        ],
    )(inputs, embedding, emb_t)
    denom = jnp.float32(n * dim)
    per_level = jnp.sum(loss, axis=1) / denom
    cb = per_level[0] + per_level[1] + per_level[2] + per_level[3]
    commit = cb
    vq = cb + jnp.float32(_COMMIT) * commit
    return (q, idx, vq, cb, commit)


# speedup vs baseline: 1.7365x; 1.4568x over previous
import jax
import jax.numpy as jnp
from jax.experimental import pallas as pl
from jax.experimental.pallas import tpu as pltpu

_LEVELS = 4
_CODES = 1024
_DIM = 64
_COMMIT = 0.25
_BLOCK = 512
_BIG = 2 ** 30


def _foldsum(s):
    acc = s[0:8, :]
    for j in range(1, 8):
        acc = acc + s[8 * j:8 * j + 8, :]
    acc = acc[0:4, :] + acc[4:8, :]
    acc = acc[0:2, :] + acc[2:4, :]
    return acc[0:1, :] + acc[1:2, :]


def _rvq_kernel(x_ref, emb_ref, embT_ref, q_ref, idx_ref, loss_ref,
                esqb_ref, iota_ref, split_ref, emb2_ref):
    i = pl.program_id(0)

    @pl.when(i == 0)
    def _init():
        for lvl in range(_LEVELS):
            eT = embT_ref[lvl]
            esq_row = _foldsum(eT * eT)
            esqb_ref[lvl] = jnp.broadcast_to(
                esq_row.reshape(_CODES, 1), (_CODES, _BLOCK))
            hi = eT.astype(jnp.bfloat16)
            rem = eT - hi.astype(jnp.float32)
            mid = rem.astype(jnp.bfloat16)
            lo = (rem - mid.astype(jnp.float32)).astype(jnp.bfloat16)
            split_ref[lvl] = jnp.concatenate([lo, mid, hi], axis=0)
            emb2_ref[lvl] = emb_ref[lvl] + emb_ref[lvl]
        iota_ref[...] = jax.lax.broadcasted_iota(
            jnp.int32, (_CODES, _BLOCK), 0)
        loss_ref[...] = jnp.zeros_like(loss_ref)

    xT = x_ref[...].T
    rowids = iota_ref[...]
    res = xT
    qs = jnp.zeros_like(xT)
    rsq = _foldsum(res * res)
    level_idx = []
    level_loss = []
    for lvl in range(_LEVELS):
        prod2 = jax.lax.dot_general(
            emb2_ref[lvl], res, (((1,), (0,)), ((), ())),
            preferred_element_type=jnp.float32)
        d = (esqb_ref[lvl] + rsq) - prod2
        dmin = jnp.min(d, axis=0, keepdims=True)
        idx = jnp.min(jnp.where(d == dmin, rowids, jnp.int32(_BIG)),
                      axis=0, keepdims=True)
        onehot = jnp.where(rowids == idx, jnp.float32(1.0),
                           jnp.float32(0.0)).astype(jnp.bfloat16)
        q3 = jax.lax.dot_general(
            split_ref[lvl], onehot, (((1,), (0,)), ((), ())),
            preferred_element_type=jnp.float32)
        qT = (q3[0:_DIM] + q3[_DIM:2 * _DIM]) + q3[2 * _DIM:3 * _DIM]
        res = res - qT
        qs = qs + qT
        rsq = _foldsum(res * res)
        level_idx.append(idx)
        level_loss.append(rsq)
    q_ref[...] = (xT + (qs - xT)).T
    idx_ref[...] = jnp.concatenate(level_idx, axis=0)
    loss_ref[...] += jnp.concatenate(level_loss, axis=0)


def kernel(inputs, embedding):
    n, dim = inputs.shape
    grid = n // _BLOCK
    emb_t = jnp.transpose(embedding, (0, 2, 1))
    q, idx, loss = pl.pallas_call(
        _rvq_kernel,
        grid=(grid,),
        in_specs=[
            pl.BlockSpec((_BLOCK, dim), lambda i: (i, 0)),
            pl.BlockSpec((_LEVELS, _CODES, _DIM), lambda i: (0, 0, 0)),
            pl.BlockSpec((_LEVELS, _DIM, _CODES), lambda i: (0, 0, 0)),
        ],
        out_specs=(
            pl.BlockSpec((_BLOCK, dim), lambda i: (i, 0)),
            pl.BlockSpec((_LEVELS, _BLOCK), lambda i: (0, i)),
            pl.BlockSpec((_LEVELS, _BLOCK), lambda i: (0, 0)),
        ),
        out_shape=(
            jax.ShapeDtypeStruct((n, dim), jnp.float32),
            jax.ShapeDtypeStruct((_LEVELS, n), jnp.int32),
            jax.ShapeDtypeStruct((_LEVELS, _BLOCK), jnp.float32),
        ),
        scratch_shapes=[
            pltpu.VMEM((_LEVELS, _CODES, _BLOCK), jnp.float32),
            pltpu.VMEM((_CODES, _BLOCK), jnp.int32),
            pltpu.VMEM((_LEVELS, 3 * _DIM, _CODES), jnp.bfloat16),
            pltpu.VMEM((_LEVELS, _CODES, _DIM), jnp.float32),
        ],
    )(inputs, embedding, emb_t)
    denom = jnp.float32(n * dim)
    per_level = jnp.sum(loss, axis=1) / denom
    cb = per_level[0] + per_level[1] + per_level[2] + per_level[3]
    commit = cb
    vq = cb + jnp.float32(_COMMIT) * commit
    return (q, idx, vq, cb, commit)


# speedup vs baseline: 2.1276x; 1.2252x over previous
import jax
import jax.numpy as jnp
from jax.experimental import pallas as pl
from jax.experimental.pallas import tpu as pltpu

_LEVELS = 4
_CODES = 1024
_DIM = 64
_COMMIT = 0.25
_BLOCK = 1024
_BIG = 2 ** 30


def _foldsum(s):
    acc = s[0:8, :]
    for j in range(1, 8):
        acc = acc + s[8 * j:8 * j + 8, :]
    acc = acc[0:4, :] + acc[4:8, :]
    acc = acc[0:2, :] + acc[2:4, :]
    return acc[0:1, :] + acc[1:2, :]


def _rvq_kernel(x_ref, emb_ref, embT_ref, q_ref, idx_ref, loss_ref,
                esqb_ref, iota_ref, split_ref, emb2_ref):
    i = pl.program_id(0)

    @pl.when(i == 0)
    def _init():
        for lvl in range(_LEVELS):
            eT = embT_ref[lvl]
            esq_row = _foldsum(eT * eT)
            esqb_ref[lvl] = jnp.broadcast_to(
                esq_row.reshape(_CODES, 1), (_CODES, _BLOCK))
            hi = eT.astype(jnp.bfloat16)
            rem = eT - hi.astype(jnp.float32)
            mid = rem.astype(jnp.bfloat16)
            lo = (rem - mid.astype(jnp.float32)).astype(jnp.bfloat16)
            split_ref[lvl] = jnp.concatenate([lo, mid, hi], axis=0)
            emb2_ref[lvl] = emb_ref[lvl] + emb_ref[lvl]
        iota_ref[...] = jax.lax.broadcasted_iota(
            jnp.int32, (_CODES, _BLOCK), 0)
        loss_ref[...] = jnp.zeros_like(loss_ref)

    xT = x_ref[...].T
    rowids = iota_ref[...]
    res = xT
    qs = jnp.zeros_like(xT)
    rsq = _foldsum(res * res)
    level_idx = []
    level_loss = []
    for lvl in range(_LEVELS):
        prod2 = jax.lax.dot_general(
            emb2_ref[lvl], res, (((1,), (0,)), ((), ())),
            preferred_element_type=jnp.float32)
        d = (esqb_ref[lvl] + rsq) - prod2
        dmin = jnp.min(d, axis=0, keepdims=True)
        idx = jnp.min(jnp.where(d == dmin, rowids, jnp.int32(_BIG)),
                      axis=0, keepdims=True)
        onehot = jnp.where(rowids == idx, jnp.float32(1.0),
                           jnp.float32(0.0)).astype(jnp.bfloat16)
        q3 = jax.lax.dot_general(
            split_ref[lvl], onehot, (((1,), (0,)), ((), ())),
            preferred_element_type=jnp.float32)
        qT = (q3[0:_DIM] + q3[_DIM:2 * _DIM]) + q3[2 * _DIM:3 * _DIM]
        res = res - qT
        qs = qs + qT
        rsq = _foldsum(res * res)
        level_idx.append(idx)
        level_loss.append(rsq)
    q_ref[...] = (xT + (qs - xT)).T
    idx_ref[...] = jnp.concatenate(level_idx, axis=0)
    loss_ref[...] += jnp.concatenate(level_loss, axis=0)


def kernel(inputs, embedding):
    n, dim = inputs.shape
    grid = n // _BLOCK
    emb_t = jnp.transpose(embedding, (0, 2, 1))
    q, idx, loss = pl.pallas_call(
        _rvq_kernel,
        grid=(grid,),
        in_specs=[
            pl.BlockSpec((_BLOCK, dim), lambda i: (i, 0)),
            pl.BlockSpec((_LEVELS, _CODES, _DIM), lambda i: (0, 0, 0)),
            pl.BlockSpec((_LEVELS, _DIM, _CODES), lambda i: (0, 0, 0)),
        ],
        out_specs=(
            pl.BlockSpec((_BLOCK, dim), lambda i: (i, 0)),
            pl.BlockSpec((_LEVELS, _BLOCK), lambda i: (0, i)),
            pl.BlockSpec((_LEVELS, _BLOCK), lambda i: (0, 0)),
        ),
        out_shape=(
            jax.ShapeDtypeStruct((n, dim), jnp.float32),
            jax.ShapeDtypeStruct((_LEVELS, n), jnp.int32),
            jax.ShapeDtypeStruct((_LEVELS, _BLOCK), jnp.float32),
        ),
        scratch_shapes=[
            pltpu.VMEM((_LEVELS, _CODES, _BLOCK), jnp.float32),
            pltpu.VMEM((_CODES, _BLOCK), jnp.int32),
            pltpu.VMEM((_LEVELS, 3 * _DIM, _CODES), jnp.bfloat16),
            pltpu.VMEM((_LEVELS, _CODES, _DIM), jnp.float32),
        ],
    )(inputs, embedding, emb_t)
    denom = jnp.float32(n * dim)
    per_level = jnp.sum(loss, axis=1) / denom
    cb = per_level[0] + per_level[1] + per_level[2] + per_level[3]
    commit = cb
    vq = cb + jnp.float32(_COMMIT) * commit
    return (q, idx, vq, cb, commit)


# speedup vs baseline: 2.2510x; 1.0580x over previous
import jax
import jax.numpy as jnp
from jax.experimental import pallas as pl
from jax.experimental.pallas import tpu as pltpu

_LEVELS = 4
_CODES = 1024
_DIM = 64
_COMMIT = 0.25
_BLOCK = 2048
_BIG = 2 ** 30


def _foldsum(s):
    acc = s[0:8, :]
    for j in range(1, 8):
        acc = acc + s[8 * j:8 * j + 8, :]
    acc = acc[0:4, :] + acc[4:8, :]
    acc = acc[0:2, :] + acc[2:4, :]
    return acc[0:1, :] + acc[1:2, :]


def _rvq_kernel(x_ref, emb_ref, embT_ref, q_ref, idx_ref, loss_ref,
                esqb_ref, iota_ref, split_ref, emb2_ref):
    i = pl.program_id(0)

    @pl.when(i == 0)
    def _init():
        for lvl in range(_LEVELS):
            eT = embT_ref[lvl]
            esq_row = _foldsum(eT * eT)
            esqb_ref[lvl] = jnp.broadcast_to(
                esq_row.reshape(_CODES, 1), (_CODES, _BLOCK))
            hi = eT.astype(jnp.bfloat16)
            rem = eT - hi.astype(jnp.float32)
            mid = rem.astype(jnp.bfloat16)
            lo = (rem - mid.astype(jnp.float32)).astype(jnp.bfloat16)
            split_ref[lvl] = jnp.concatenate([lo, mid, hi], axis=0)
            emb2_ref[lvl] = emb_ref[lvl] + emb_ref[lvl]
        iota_ref[...] = jax.lax.broadcasted_iota(
            jnp.int32, (_CODES, _BLOCK), 0)
        loss_ref[...] = jnp.zeros_like(loss_ref)

    xT = x_ref[...].T
    rowids = iota_ref[...]
    res = xT
    qs = jnp.zeros_like(xT)
    rsq = _foldsum(res * res)
    level_idx = []
    level_loss = []
    for lvl in range(_LEVELS):
        prod2 = jax.lax.dot_general(
            emb2_ref[lvl], res, (((1,), (0,)), ((), ())),
            preferred_element_type=jnp.float32)
        d = (esqb_ref[lvl] + rsq) - prod2
        dmin = jnp.min(d, axis=0, keepdims=True)
        idx = jnp.min(jnp.where(d == dmin, rowids, jnp.int32(_BIG)),
                      axis=0, keepdims=True)
        onehot = jnp.where(rowids == idx, jnp.float32(1.0),
                           jnp.float32(0.0)).astype(jnp.bfloat16)
        q3 = jax.lax.dot_general(
            split_ref[lvl], onehot, (((1,), (0,)), ((), ())),
            preferred_element_type=jnp.float32)
        qT = (q3[0:_DIM] + q3[_DIM:2 * _DIM]) + q3[2 * _DIM:3 * _DIM]
        res = res - qT
        qs = qs + qT
        rsq = _foldsum(res * res)
        level_idx.append(idx)
        level_loss.append(rsq)
    q_ref[...] = (xT + (qs - xT)).T
    idx_ref[...] = jnp.concatenate(level_idx, axis=0)
    loss_ref[...] += jnp.concatenate(level_loss, axis=0)


def kernel(inputs, embedding):
    n, dim = inputs.shape
    grid = n // _BLOCK
    emb_t = jnp.transpose(embedding, (0, 2, 1))
    q, idx, loss = pl.pallas_call(
        _rvq_kernel,
        grid=(grid,),
        in_specs=[
            pl.BlockSpec((_BLOCK, dim), lambda i: (i, 0)),
            pl.BlockSpec((_LEVELS, _CODES, _DIM), lambda i: (0, 0, 0)),
            pl.BlockSpec((_LEVELS, _DIM, _CODES), lambda i: (0, 0, 0)),
        ],
        out_specs=(
            pl.BlockSpec((_BLOCK, dim), lambda i: (i, 0)),
            pl.BlockSpec((_LEVELS, _BLOCK), lambda i: (0, i)),
            pl.BlockSpec((_LEVELS, _BLOCK), lambda i: (0, 0)),
        ),
        out_shape=(
            jax.ShapeDtypeStruct((n, dim), jnp.float32),
            jax.ShapeDtypeStruct((_LEVELS, n), jnp.int32),
            jax.ShapeDtypeStruct((_LEVELS, _BLOCK), jnp.float32),
        ),
        scratch_shapes=[
            pltpu.VMEM((_LEVELS, _CODES, _BLOCK), jnp.float32),
            pltpu.VMEM((_CODES, _BLOCK), jnp.int32),
            pltpu.VMEM((_LEVELS, 3 * _DIM, _CODES), jnp.bfloat16),
            pltpu.VMEM((_LEVELS, _CODES, _DIM), jnp.float32),
        ],
    )(inputs, embedding, emb_t)
    denom = jnp.float32(n * dim)
    per_level = jnp.sum(loss, axis=1) / denom
    cb = per_level[0] + per_level[1] + per_level[2] + per_level[3]
    commit = cb
    vq = cb + jnp.float32(_COMMIT) * commit
    return (q, idx, vq, cb, commit)


# speedup vs baseline: 2.3936x; 1.0634x over previous
import jax
import jax.numpy as jnp
from jax.experimental import pallas as pl
from jax.experimental.pallas import tpu as pltpu

_LEVELS = 4
_CODES = 1024
_DIM = 64
_COMMIT = 0.25
_BLOCK = 2048
_BIG = 2 ** 30


def _foldsum(s):
    acc = s[0:8, :]
    for j in range(1, 8):
        acc = acc + s[8 * j:8 * j + 8, :]
    acc = acc[0:4, :] + acc[4:8, :]
    acc = acc[0:2, :] + acc[2:4, :]
    return acc[0:1, :] + acc[1:2, :]


def _rvq_kernel(x_ref, emb_ref, embT_ref, q_ref, idx_ref, loss_ref,
                esqb_ref, iota_ref, split_ref, emb2_ref):
    i = pl.program_id(0)

    @pl.when(i == 0)
    def _init():
        for lvl in range(_LEVELS):
            eT = embT_ref[lvl]
            esq_row = _foldsum(eT * eT)
            esqb_ref[lvl] = jnp.broadcast_to(
                esq_row.reshape(_CODES, 1), (_CODES, _BLOCK))
            hi = eT.astype(jnp.bfloat16)
            rem = eT - hi.astype(jnp.float32)
            mid = rem.astype(jnp.bfloat16)
            lo = (rem - mid.astype(jnp.float32)).astype(jnp.bfloat16)
            split_ref[lvl] = jnp.concatenate([lo, mid, hi], axis=0)
            emb2_ref[lvl] = emb_ref[lvl] + emb_ref[lvl]
        iota_ref[...] = jax.lax.broadcasted_iota(
            jnp.int32, (_CODES, _BLOCK), 0).astype(jnp.float32)
        loss_ref[...] = jnp.zeros_like(loss_ref)

    xT = x_ref[...].T
    rowids = iota_ref[...]
    res = xT
    qs = jnp.zeros_like(xT)
    rsq = _foldsum(res * res)
    level_idx = []
    level_loss = []
    for lvl in range(_LEVELS):
        prod2 = jax.lax.dot_general(
            emb2_ref[lvl], res, (((1,), (0,)), ((), ())),
            preferred_element_type=jnp.float32)
        d = (esqb_ref[lvl] + rsq) - prod2
        dmin = jnp.min(d, axis=0, keepdims=True)
        idx = jnp.min(jnp.where(d == dmin, rowids, jnp.float32(_BIG)),
                      axis=0, keepdims=True)
        onehot = jnp.where(rowids == idx, jnp.float32(1.0),
                           jnp.float32(0.0)).astype(jnp.bfloat16)
        q3 = jax.lax.dot_general(
            split_ref[lvl], onehot, (((1,), (0,)), ((), ())),
            preferred_element_type=jnp.float32)
        qT = (q3[0:_DIM] + q3[_DIM:2 * _DIM]) + q3[2 * _DIM:3 * _DIM]
        res = res - qT
        qs = qs + qT
        rsq = _foldsum(res * res)
        level_idx.append(idx.astype(jnp.int32))
        level_loss.append(rsq)
    q_ref[...] = (xT + (qs - xT)).T
    idx_ref[...] = jnp.concatenate(level_idx, axis=0)
    loss_ref[...] += jnp.concatenate(level_loss, axis=0)


def kernel(inputs, embedding):
    n, dim = inputs.shape
    grid = n // _BLOCK
    emb_t = jnp.transpose(embedding, (0, 2, 1))
    q, idx, loss = pl.pallas_call(
        _rvq_kernel,
        grid=(grid,),
        in_specs=[
            pl.BlockSpec((_BLOCK, dim), lambda i: (i, 0)),
            pl.BlockSpec((_LEVELS, _CODES, _DIM), lambda i: (0, 0, 0)),
            pl.BlockSpec((_LEVELS, _DIM, _CODES), lambda i: (0, 0, 0)),
        ],
        out_specs=(
            pl.BlockSpec((_BLOCK, dim), lambda i: (i, 0)),
            pl.BlockSpec((_LEVELS, _BLOCK), lambda i: (0, i)),
            pl.BlockSpec((_LEVELS, _BLOCK), lambda i: (0, 0)),
        ),
        out_shape=(
            jax.ShapeDtypeStruct((n, dim), jnp.float32),
            jax.ShapeDtypeStruct((_LEVELS, n), jnp.int32),
            jax.ShapeDtypeStruct((_LEVELS, _BLOCK), jnp.float32),
        ),
        scratch_shapes=[
            pltpu.VMEM((_LEVELS, _CODES, _BLOCK), jnp.float32),
            pltpu.VMEM((_CODES, _BLOCK), jnp.float32),
            pltpu.VMEM((_LEVELS, 3 * _DIM, _CODES), jnp.bfloat16),
            pltpu.VMEM((_LEVELS, _CODES, _DIM), jnp.float32),
        ],
    )(inputs, embedding, emb_t)
    denom = jnp.float32(n * dim)
    per_level = jnp.sum(loss, axis=1) / denom
    cb = per_level[0] + per_level[1] + per_level[2] + per_level[3]
    commit = cb
    vq = cb + jnp.float32(_COMMIT) * commit
    return (q, idx, vq, cb, commit)


# speedup vs baseline: 2.4593x; 1.0275x over previous
import jax
import jax.numpy as jnp
from jax.experimental import pallas as pl
from jax.experimental.pallas import tpu as pltpu

_LEVELS = 4
_CODES = 1024
_DIM = 64
_COMMIT = 0.25
_BLOCK = 2048
_BIG = 2 ** 30


def _foldsum(s):
    acc = s[0:8, :]
    for j in range(1, 8):
        acc = acc + s[8 * j:8 * j + 8, :]
    acc = acc[0:4, :] + acc[4:8, :]
    acc = acc[0:2, :] + acc[2:4, :]
    return acc[0:1, :] + acc[1:2, :]


def _rvq_kernel(x_ref, emb_ref, embT_ref, q_ref, idx_ref, loss_ref,
                esqb_ref, iota_ref, split_ref, emb2_ref):
    i = pl.program_id(0)

    @pl.when(i == 0)
    def _init():
        for lvl in range(_LEVELS):
            eT = embT_ref[lvl]
            esq_row = _foldsum(eT * eT)
            esqb_ref[lvl] = esq_row.reshape(_CODES, 1)
            hi = eT.astype(jnp.bfloat16)
            rem = eT - hi.astype(jnp.float32)
            mid = rem.astype(jnp.bfloat16)
            lo = (rem - mid.astype(jnp.float32)).astype(jnp.bfloat16)
            split_ref[lvl] = jnp.concatenate([lo, mid, hi], axis=0)
            emb2_ref[lvl] = emb_ref[lvl] + emb_ref[lvl]
        iota_ref[...] = jax.lax.broadcasted_iota(
            jnp.int32, (_CODES, _BLOCK), 0).astype(jnp.float32)
        loss_ref[...] = jnp.zeros_like(loss_ref)

    xT = x_ref[...].T
    rowids = iota_ref[...]
    res = xT
    qs = jnp.zeros_like(xT)
    rsq = _foldsum(res * res)
    level_idx = []
    level_loss = []
    for lvl in range(_LEVELS):
        prod2 = jax.lax.dot_general(
            emb2_ref[lvl], res, (((1,), (0,)), ((), ())),
            preferred_element_type=jnp.float32)
        d = (esqb_ref[lvl] + rsq) - prod2
        dmin = jnp.min(d, axis=0, keepdims=True)
        idx = jnp.min(jnp.where(d == dmin, rowids, jnp.float32(_BIG)),
                      axis=0, keepdims=True)
        onehot = jnp.where(rowids == idx, jnp.float32(1.0),
                           jnp.float32(0.0)).astype(jnp.bfloat16)
        q3 = jax.lax.dot_general(
            split_ref[lvl], onehot, (((1,), (0,)), ((), ())),
            preferred_element_type=jnp.float32)
        qT = (q3[0:_DIM] + q3[_DIM:2 * _DIM]) + q3[2 * _DIM:3 * _DIM]
        res = res - qT
        qs = qs + qT
        rsq = _foldsum(res * res)
        level_idx.append(idx.astype(jnp.int32))
        level_loss.append(rsq)
    q_ref[...] = (xT + (qs - xT)).T
    idx_ref[...] = jnp.concatenate(level_idx, axis=0)
    loss_ref[...] += jnp.concatenate(level_loss, axis=0)


def kernel(inputs, embedding):
    n, dim = inputs.shape
    grid = n // _BLOCK
    emb_t = jnp.transpose(embedding, (0, 2, 1))
    q, idx, loss = pl.pallas_call(
        _rvq_kernel,
        grid=(grid,),
        in_specs=[
            pl.BlockSpec((_BLOCK, dim), lambda i: (i, 0)),
            pl.BlockSpec((_LEVELS, _CODES, _DIM), lambda i: (0, 0, 0)),
            pl.BlockSpec((_LEVELS, _DIM, _CODES), lambda i: (0, 0, 0)),
        ],
        out_specs=(
            pl.BlockSpec((_BLOCK, dim), lambda i: (i, 0)),
            pl.BlockSpec((_LEVELS, _BLOCK), lambda i: (0, i)),
            pl.BlockSpec((_LEVELS, _BLOCK), lambda i: (0, 0)),
        ),
        out_shape=(
            jax.ShapeDtypeStruct((n, dim), jnp.float32),
            jax.ShapeDtypeStruct((_LEVELS, n), jnp.int32),
            jax.ShapeDtypeStruct((_LEVELS, _BLOCK), jnp.float32),
        ),
        scratch_shapes=[
            pltpu.VMEM((_LEVELS, _CODES, 1), jnp.float32),
            pltpu.VMEM((_CODES, _BLOCK), jnp.float32),
            pltpu.VMEM((_LEVELS, 3 * _DIM, _CODES), jnp.bfloat16),
            pltpu.VMEM((_LEVELS, _CODES, _DIM), jnp.float32),
        ],
    )(inputs, embedding, emb_t)
    denom = jnp.float32(n * dim)
    per_level = jnp.sum(loss, axis=1) / denom
    cb = per_level[0] + per_level[1] + per_level[2] + per_level[3]
    commit = cb
    vq = cb + jnp.float32(_COMMIT) * commit
    return (q, idx, vq, cb, commit)
